# baseline (speedup 1.0000x reference)
import jax
import jax.numpy as jnp
from jax.experimental import pallas as pl
from jax.experimental.pallas import tpu as pltpu

_N = 207
_U = 64
_SEQ = 12
_HOR = 12
_B = 64
_K = 2
_NMAT = _K + 1
_NC = 2
_BC = _B // _NC


def _pack(a2d, b2d):
    return jnp.concatenate(
        [a2d.reshape(_N, _BC, _U), b2d.reshape(_N, _BC, _U)], axis=-1
    )


def _fmm(z, rr, w):
    o = w.shape[-1]
    acc = jnp.dot(
        z.reshape(_N * _BC, _U), w[:_U], preferred_element_type=jnp.float32
    )
    acc += jnp.dot(
        rr.reshape(_N * _BC, 2 * _U), w[_U:], preferred_element_type=jnp.float32
    )
    return acc.reshape(_N, _BC, o)


def _candidate(h, rh, xc, support, wch, bc, r_ref):
    z2d = rh.reshape(_N, _BC * _U)
    z1 = jnp.dot(support, z2d, preferred_element_type=jnp.float32)
    z2 = jnp.dot(support, z1, preferred_element_type=jnp.float32)
    r_ref[...] = _pack(z1, z2)
    return jnp.tanh(_fmm(rh, r_ref[...], wch) + xc + bc.reshape(1, 1, -1))


def _x1_apply(xs, w3):
    return (
        xs[0][:, :, None] * w3[0].reshape(1, 1, -1)
        + xs[1][:, :, None] * w3[1].reshape(1, 1, -1)
        + xs[2][:, :, None] * w3[2].reshape(1, 1, -1)
    )


def _step(x, hh_ref, support, r_ref, x_ref, w0, w1):
    w0gx, w0gh, w0bg, w0cx, w0ch, w0bc = w0
    w1gcx, w1gh, w1bg, w1ch, w1bc = w1
    hh = hh_ref[...]
    h0 = hh[:, :, :_U]
    h1 = hh[:, :, _U:]

    zx = jnp.concatenate([x, h0.reshape(_N, _BC * _U)], axis=-1)
    d1 = jnp.dot(support, zx, preferred_element_type=jnp.float32)
    d2 = jnp.dot(support, d1, preferred_element_type=jnp.float32)
    x_terms = (x, d1[:, :_BC], d2[:, :_BC])
    r_ref[...] = _pack(d1[:, _BC:], d2[:, _BC:])
    g = jax.nn.sigmoid(
        _fmm(h0, r_ref[...], w0gh)
        + _x1_apply(x_terms, w0gx)
        + w0bg.reshape(1, 1, -1)
    )
    r = g[:, :, :_U]
    u = g[:, :, _U:]
    c = _candidate(h0, r * h0, _x1_apply(x_terms, w0cx), support,
                   w0ch, w0bc, r_ref)
    h0n = u * h0 + (1.0 - u) * c

    zz = jnp.concatenate(
        [h0n.reshape(_N, _BC * _U), h1.reshape(_N, _BC * _U)], axis=-1
    )
    e1 = jnp.dot(support, zz, preferred_element_type=jnp.float32)
    e2 = jnp.dot(support, e1, preferred_element_type=jnp.float32)
    half = _BC * _U
    x_ref[...] = _pack(e1[:, :half], e2[:, :half])
    r_ref[...] = _pack(e1[:, half:], e2[:, half:])
    xgc = _fmm(h0n, x_ref[...], w1gcx)
    g = jax.nn.sigmoid(
        _fmm(h1, r_ref[...], w1gh)
        + xgc[:, :, : 2 * _U]
        + w1bg.reshape(1, 1, -1)
    )
    r = g[:, :, :_U]
    u = g[:, :, _U:]
    c = _candidate(h1, r * h1, xgc[:, :, 2 * _U :], support,
                   w1ch, w1bc, r_ref)
    h1n = u * h1 + (1.0 - u) * c

    hh_ref[...] = jnp.concatenate([h0n, h1n], axis=-1)
    return h1n


def _dcgru_kernel(
    xt_ref, adj_ref,
    e0gx_ref, e0gh_ref, e0bg_ref, e0cx_ref, e0ch_ref, e0bc_ref,
    e1gcx_ref, e1gh_ref, e1bg_ref, e1ch_ref, e1bc_ref,
    d0gx_ref, d0gh_ref, d0bg_ref, d0cx_ref, d0ch_ref, d0bc_ref,
    d1gcx_ref, d1gh_ref, d1bg_ref, d1ch_ref, d1bc_ref,
    wp_ref, bp_ref,
    out_ref,
    hh_ref, di_ref, r_ref, x_ref,
):
    adj = adj_ref[...]
    support = adj / (jnp.sum(adj, axis=1, keepdims=True) + 1e-8)

    hh_ref[...] = jnp.zeros((_N, _BC, 2 * _U), jnp.float32)

    ew0 = (e0gx_ref[...], e0gh_ref[...], e0bg_ref[...],
           e0cx_ref[...], e0ch_ref[...], e0bc_ref[...])
    ew1 = (e1gcx_ref[...], e1gh_ref[...], e1bg_ref[...],
           e1ch_ref[...], e1bc_ref[...])

    def enc_body(t, carry):
        _step(xt_ref[0, t], hh_ref, support, r_ref, x_ref, ew0, ew1)
        return carry

    jax.lax.fori_loop(0, _SEQ, enc_body, 0)

    dw0 = (d0gx_ref[...], d0gh_ref[...], d0bg_ref[...],
           d0cx_ref[...], d0ch_ref[...], d0bc_ref[...])
    dw1 = (d1gcx_ref[...], d1gh_ref[...], d1bg_ref[...],
           d1ch_ref[...], d1bc_ref[...])
    wp = wp_ref[...]
    bp = bp_ref[0, 0]

    di_ref[...] = jnp.zeros((_N, _BC), jnp.float32)

    def dec_body(t, carry):
        h1n = _step(di_ref[...], hh_ref, support, r_ref, x_ref, dw0, dw1)
        proj = jnp.sum(h1n * wp.reshape(1, 1, _U), axis=-1) + bp
        out_ref[0, t] = proj
        di_ref[...] = proj
        return carry

    jax.lax.fori_loop(0, _HOR, dec_body, 0)


def _split_w(w, fx):
    f = fx + _U
    wx = jnp.concatenate([w[k * f : k * f + fx] for k in range(_NMAT)], axis=0)
    wh = jnp.concatenate([w[k * f + fx : (k + 1) * f] for k in range(_NMAT)], axis=0)
    return wx, wh


def _full_spec(shape):
    return pl.BlockSpec(shape, lambda c: (0,) * len(shape))


@jax.jit
def kernel(inputs, adj,
           enc_Wg0, enc_bg0, enc_Wc0, enc_bc0,
           enc_Wg1, enc_bg1, enc_Wc1, enc_bc1,
           dec_Wg0, dec_bg0, dec_Wc0, dec_bc0,
           dec_Wg1, dec_bg1, dec_Wc1, dec_bc1,
           Wp, bp):
    xt = jnp.transpose(
        inputs.reshape(_SEQ, _B, _N), (0, 2, 1)
    ).reshape(_SEQ, _N, _NC, _BC)
    xt = jnp.transpose(xt, (2, 0, 1, 3))

    e0gx, e0gh = _split_w(enc_Wg0, 1)
    e0cx, e0ch = _split_w(enc_Wc0, 1)
    e1gx, e1gh = _split_w(enc_Wg1, _U)
    e1cx, e1ch = _split_w(enc_Wc1, _U)
    d0gx, d0gh = _split_w(dec_Wg0, 1)
    d0cx, d0ch = _split_w(dec_Wc0, 1)
    d1gx, d1gh = _split_w(dec_Wg1, _U)
    d1cx, d1ch = _split_w(dec_Wc1, _U)
    e1gcx = jnp.concatenate([e1gx, e1cx], axis=1)
    d1gcx = jnp.concatenate([d1gx, d1cx], axis=1)

    operands = (
        xt, adj,
        e0gx, e0gh, enc_bg0.reshape(1, -1), e0cx, e0ch, enc_bc0.reshape(1, -1),
        e1gcx, e1gh, enc_bg1.reshape(1, -1), e1ch, enc_bc1.reshape(1, -1),
        d0gx, d0gh, dec_bg0.reshape(1, -1), d0cx, d0ch, dec_bc0.reshape(1, -1),
        d1gcx, d1gh, dec_bg1.reshape(1, -1), d1ch, dec_bc1.reshape(1, -1),
        Wp.reshape(1, _U), bp.reshape(1, 1),
    )

    in_specs = [
        pl.BlockSpec((1, _SEQ, _N, _BC), lambda c: (c, 0, 0, 0)),
    ] + [_full_spec(op.shape) for op in operands[1:]]

    out = pl.pallas_call(
        _dcgru_kernel,
        grid=(_NC,),
        out_shape=jax.ShapeDtypeStruct((_NC, _HOR, _N, _BC), jnp.float32),
        in_specs=in_specs,
        out_specs=pl.BlockSpec((1, _HOR, _N, _BC), lambda c: (c, 0, 0, 0)),
        compiler_params=pltpu.CompilerParams(
            vmem_limit_bytes=64 * 1024 * 1024,
        ),
        scratch_shapes=[
            pltpu.VMEM((_N, _BC, 2 * _U), jnp.float32),
            pltpu.VMEM((_N, _BC), jnp.float32),
            pltpu.VMEM((_N, _BC, 2 * _U), jnp.float32),
            pltpu.VMEM((_N, _BC, 2 * _U), jnp.float32),
        ],
    )(*operands)
    return jnp.transpose(out, (1, 0, 3, 2)).reshape(_HOR, _B, _N)


# speedup vs baseline: 1.2430x; 1.2430x over previous
import jax
import jax.numpy as jnp
from jax.experimental import pallas as pl
from jax.experimental.pallas import tpu as pltpu

_N = 207
_U = 64
_SEQ = 12
_HOR = 12
_B = 64
_K = 2
_NMAT = _K + 1
_NC = 2
_BC = _B // _NC


def _pack(a2d, b2d):
    return jnp.concatenate(
        [a2d.reshape(_N, _BC, _U), b2d.reshape(_N, _BC, _U)], axis=-1
    )


def _bf(v):
    return v.astype(jnp.bfloat16)


def _fmm(z, rr, w):
    o = w.shape[-1]
    acc = jnp.dot(
        _bf(z).reshape(_N * _BC, _U), w[:_U],
        preferred_element_type=jnp.float32,
    )
    acc += jnp.dot(
        rr.reshape(_N * _BC, 2 * _U), w[_U:], preferred_element_type=jnp.float32
    )
    return acc.reshape(_N, _BC, o)


def _candidate(h, rh, xc, support, wch, bc, r_ref):
    z2d = _bf(rh).reshape(_N, _BC * _U)
    z1 = _bf(jnp.dot(support, z2d, preferred_element_type=jnp.float32))
    z2 = _bf(jnp.dot(support, z1, preferred_element_type=jnp.float32))
    r_ref[...] = _pack(z1, z2)
    return jnp.tanh(_fmm(rh, r_ref[...], wch) + xc + bc.reshape(1, 1, -1))


def _x1_apply(xs, w3):
    return (
        xs[0][:, :, None] * w3[0].reshape(1, 1, -1)
        + xs[1][:, :, None] * w3[1].reshape(1, 1, -1)
        + xs[2][:, :, None] * w3[2].reshape(1, 1, -1)
    )


def _step(x, hh_ref, support, r_ref, x_ref, w0, w1):
    w0gx, w0gh, w0bg, w0cx, w0ch, w0bc = w0
    w1gcx, w1gh, w1bg, w1ch, w1bc = w1
    hh = hh_ref[...]
    h0 = hh[:, :, :_U]
    h1 = hh[:, :, _U:]

    zx = jnp.concatenate([_bf(x), _bf(h0).reshape(_N, _BC * _U)], axis=-1)
    d1 = _bf(jnp.dot(support, zx, preferred_element_type=jnp.float32))
    d2 = _bf(jnp.dot(support, d1, preferred_element_type=jnp.float32))
    x_terms = (x, d1[:, :_BC].astype(jnp.float32),
               d2[:, :_BC].astype(jnp.float32))
    r_ref[...] = _pack(d1[:, _BC:], d2[:, _BC:])
    g = jax.nn.sigmoid(
        _fmm(h0, r_ref[...], w0gh)
        + _x1_apply(x_terms, w0gx)
        + w0bg.reshape(1, 1, -1)
    )
    r = g[:, :, :_U]
    u = g[:, :, _U:]
    c = _candidate(h0, r * h0, _x1_apply(x_terms, w0cx), support,
                   w0ch, w0bc, r_ref)
    h0n = u * h0 + (1.0 - u) * c

    zz = jnp.concatenate(
        [_bf(h0n).reshape(_N, _BC * _U), _bf(h1).reshape(_N, _BC * _U)],
        axis=-1,
    )
    e1 = _bf(jnp.dot(support, zz, preferred_element_type=jnp.float32))
    e2 = _bf(jnp.dot(support, e1, preferred_element_type=jnp.float32))
    half = _BC * _U
    x_ref[...] = _pack(e1[:, :half], e2[:, :half])
    r_ref[...] = _pack(e1[:, half:], e2[:, half:])
    xgc = _fmm(h0n, x_ref[...], w1gcx)
    g = jax.nn.sigmoid(
        _fmm(h1, r_ref[...], w1gh)
        + xgc[:, :, : 2 * _U]
        + w1bg.reshape(1, 1, -1)
    )
    r = g[:, :, :_U]
    u = g[:, :, _U:]
    c = _candidate(h1, r * h1, xgc[:, :, 2 * _U :], support,
                   w1ch, w1bc, r_ref)
    h1n = u * h1 + (1.0 - u) * c

    hh_ref[...] = jnp.concatenate([h0n, h1n], axis=-1)
    return h1n


def _dcgru_kernel(
    xt_ref, adj_ref,
    e0gx_ref, e0gh_ref, e0bg_ref, e0cx_ref, e0ch_ref, e0bc_ref,
    e1gcx_ref, e1gh_ref, e1bg_ref, e1ch_ref, e1bc_ref,
    d0gx_ref, d0gh_ref, d0bg_ref, d0cx_ref, d0ch_ref, d0bc_ref,
    d1gcx_ref, d1gh_ref, d1bg_ref, d1ch_ref, d1bc_ref,
    wp_ref, bp_ref,
    out_ref,
    hh_ref, di_ref, r_ref, x_ref,
):
    adj = adj_ref[...]
    support = (adj / (jnp.sum(adj, axis=1, keepdims=True) + 1e-8)).astype(
        jnp.bfloat16
    )

    hh_ref[...] = jnp.zeros((_N, _BC, 2 * _U), jnp.float32)

    ew0 = (e0gx_ref[...], e0gh_ref[...], e0bg_ref[...],
           e0cx_ref[...], e0ch_ref[...], e0bc_ref[...])
    ew1 = (e1gcx_ref[...], e1gh_ref[...], e1bg_ref[...],
           e1ch_ref[...], e1bc_ref[...])

    def enc_body(t, carry):
        _step(xt_ref[0, t], hh_ref, support, r_ref, x_ref, ew0, ew1)
        return carry

    jax.lax.fori_loop(0, _SEQ, enc_body, 0)

    dw0 = (d0gx_ref[...], d0gh_ref[...], d0bg_ref[...],
           d0cx_ref[...], d0ch_ref[...], d0bc_ref[...])
    dw1 = (d1gcx_ref[...], d1gh_ref[...], d1bg_ref[...],
           d1ch_ref[...], d1bc_ref[...])
    wp = wp_ref[...]
    bp = bp_ref[0, 0]

    di_ref[...] = jnp.zeros((_N, _BC), jnp.float32)

    def dec_body(t, carry):
        h1n = _step(di_ref[...], hh_ref, support, r_ref, x_ref, dw0, dw1)
        proj = jnp.sum(h1n * wp.reshape(1, 1, _U), axis=-1) + bp
        out_ref[0, t] = proj
        di_ref[...] = proj
        return carry

    jax.lax.fori_loop(0, _HOR, dec_body, 0)


def _split_w(w, fx):
    f = fx + _U
    wx = jnp.concatenate([w[k * f : k * f + fx] for k in range(_NMAT)], axis=0)
    wh = jnp.concatenate([w[k * f + fx : (k + 1) * f] for k in range(_NMAT)], axis=0)
    return wx, wh


def _full_spec(shape):
    return pl.BlockSpec(shape, lambda c: (0,) * len(shape))


@jax.jit
def kernel(inputs, adj,
           enc_Wg0, enc_bg0, enc_Wc0, enc_bc0,
           enc_Wg1, enc_bg1, enc_Wc1, enc_bc1,
           dec_Wg0, dec_bg0, dec_Wc0, dec_bc0,
           dec_Wg1, dec_bg1, dec_Wc1, dec_bc1,
           Wp, bp):
    xt = jnp.transpose(
        inputs.reshape(_SEQ, _B, _N), (0, 2, 1)
    ).reshape(_SEQ, _N, _NC, _BC)
    xt = jnp.transpose(xt, (2, 0, 1, 3))

    e0gx, e0gh = _split_w(enc_Wg0, 1)
    e0cx, e0ch = _split_w(enc_Wc0, 1)
    e1gx, e1gh = _split_w(enc_Wg1, _U)
    e1cx, e1ch = _split_w(enc_Wc1, _U)
    d0gx, d0gh = _split_w(dec_Wg0, 1)
    d0cx, d0ch = _split_w(dec_Wc0, 1)
    d1gx, d1gh = _split_w(dec_Wg1, _U)
    d1cx, d1ch = _split_w(dec_Wc1, _U)
    e1gcx = jnp.concatenate([e1gx, e1cx], axis=1)
    d1gcx = jnp.concatenate([d1gx, d1cx], axis=1)
    e0gh = e0gh.astype(jnp.bfloat16)
    e0ch = e0ch.astype(jnp.bfloat16)
    e1gcx = e1gcx.astype(jnp.bfloat16)
    e1gh = e1gh.astype(jnp.bfloat16)
    e1ch = e1ch.astype(jnp.bfloat16)
    d0gh = d0gh.astype(jnp.bfloat16)
    d0ch = d0ch.astype(jnp.bfloat16)
    d1gcx = d1gcx.astype(jnp.bfloat16)
    d1gh = d1gh.astype(jnp.bfloat16)
    d1ch = d1ch.astype(jnp.bfloat16)

    operands = (
        xt, adj,
        e0gx, e0gh, enc_bg0.reshape(1, -1), e0cx, e0ch, enc_bc0.reshape(1, -1),
        e1gcx, e1gh, enc_bg1.reshape(1, -1), e1ch, enc_bc1.reshape(1, -1),
        d0gx, d0gh, dec_bg0.reshape(1, -1), d0cx, d0ch, dec_bc0.reshape(1, -1),
        d1gcx, d1gh, dec_bg1.reshape(1, -1), d1ch, dec_bc1.reshape(1, -1),
        Wp.reshape(1, _U), bp.reshape(1, 1),
    )

    in_specs = [
        pl.BlockSpec((1, _SEQ, _N, _BC), lambda c: (c, 0, 0, 0)),
    ] + [_full_spec(op.shape) for op in operands[1:]]

    out = pl.pallas_call(
        _dcgru_kernel,
        grid=(_NC,),
        out_shape=jax.ShapeDtypeStruct((_NC, _HOR, _N, _BC), jnp.float32),
        in_specs=in_specs,
        out_specs=pl.BlockSpec((1, _HOR, _N, _BC), lambda c: (c, 0, 0, 0)),
        compiler_params=pltpu.CompilerParams(
            vmem_limit_bytes=64 * 1024 * 1024,
        ),
        scratch_shapes=[
            pltpu.VMEM((_N, _BC, 2 * _U), jnp.float32),
            pltpu.VMEM((_N, _BC), jnp.float32),
            pltpu.VMEM((_N, _BC, 2 * _U), jnp.bfloat16),
            pltpu.VMEM((_N, _BC, 2 * _U), jnp.bfloat16),
        ],
    )(*operands)
    return jnp.transpose(out, (1, 0, 3, 2)).reshape(_HOR, _B, _N)
